# baseline (speedup 1.0000x reference)
import jax
import jax.numpy as jnp
import numpy as np
from jax import lax
from jax.experimental import pallas as pl
from jax.experimental.pallas import tpu as pltpu
from jax.experimental.pallas import tpu_sc as plsc

_N = 10000
_E = 320000
_D = 128
_CH = 100
_NCHUNK = _E // _CH
_CPT = _NCHUNK // 32
_SLAB = _N // 16
_NP = 10240
_DSLAB = _NP // 16
_NGRP_FULL = _CH // 16
_TAIL = _CH - 16 * _NGRP_FULL

_CHP = 500
_NCHP = _E // _CHP
_GSP = list(range(0, _CHP - 15, 16)) + ([_CHP - 16] if _CHP % 16 else [])


def _mesh():
    return plsc.VectorSubcoreMesh(core_axis_name="c", subcore_axis_name="s")


def _rsqrt16(d):
    pos = d > 0.0
    dc = jnp.where(pos, d, 1.0)
    i = lax.bitcast_convert_type(dc, jnp.int32)
    i = jnp.int32(0x5F3759DF) - lax.shift_right_arithmetic(i, 1)
    y = lax.bitcast_convert_type(i, jnp.float32)
    for _ in range(3):
        y = y * (1.5 - 0.5 * dc * y * y)
    return jnp.where(pos, y, 0.0)


def _prep_body(srcN, dstN, wN, zeros, norm_out, accD, dinvS, dinvb,
               srcb, dstb, wb, wzb, dbuf, obuf, nb):
    c = lax.axis_index("c")
    s = lax.axis_index("s")
    t = c * 16 + s

    pltpu.sync_copy(zeros, accD.at[pl.ds(s * _DSLAB, _DSLAB)])
    plsc.subcore_barrier()

    def dchunk(i, carry):
        cid = s * (_NCHP // 16) + i
        pltpu.sync_copy(srcN.at[cid], srcb)
        pltpu.sync_copy(dstN.at[cid], dstb)
        pltpu.sync_copy(wN.at[cid], wb)
        for gs in _GSP:
            sl = pl.ds(gs, 16)
            wzb[sl] = jnp.where(srcb[sl] == dstb[sl], 0.0, wb[sl])
        pltpu.sync_copy(wzb, accD.at[srcb], add=True)
        return carry

    lax.fori_loop(0, _NCHP // 16, dchunk, 0)
    plsc.subcore_barrier()

    pltpu.sync_copy(accD.at[pl.ds(s * _DSLAB, _DSLAB)], dbuf)
    for k in range(_DSLAB // 16):
        sl = pl.ds(k * 16, 16)
        obuf[sl] = _rsqrt16(dbuf[sl])
    pltpu.sync_copy(obuf, dinvS.at[pl.ds(s * _DSLAB, _DSLAB)])
    plsc.subcore_barrier()

    pltpu.sync_copy(dinvS, dinvb)

    def nchunk(i, carry):
        cid = t * (_NCHP // 32) + i
        pltpu.sync_copy(srcN.at[cid], srcb)
        pltpu.sync_copy(dstN.at[cid], dstb)
        pltpu.sync_copy(wN.at[cid], wb)
        for gs in _GSP:
            sl = pl.ds(gs, 16)
            sv = srcb[sl]
            dv = dstb[sl]
            wz = jnp.where(sv == dv, 0.0, wb[sl])
            ds_ = plsc.load_gather(dinvb, [sv])
            dd_ = plsc.load_gather(dinvb, [dv])
            nb[sl] = -(ds_ * wz * dd_)
        pltpu.sync_copy(nb, norm_out.at[cid])
        return carry

    lax.fori_loop(0, _NCHP // 32, nchunk, 0)


def _run_prep(srcN, dstN, wN, zeros):
    f = pl.kernel(
        _prep_body,
        out_type=jax.ShapeDtypeStruct((_NCHP, _CHP), jnp.float32),
        mesh=_mesh(),
        compiler_params=pltpu.CompilerParams(needs_layout_passes=False),
        scratch_types=[
            pltpu.VMEM_SHARED((_NP,), jnp.float32),
            pltpu.VMEM_SHARED((_NP,), jnp.float32),
            pltpu.VMEM((_NP,), jnp.float32),
            pltpu.VMEM((_CHP,), jnp.int32),
            pltpu.VMEM((_CHP,), jnp.int32),
            pltpu.VMEM((_CHP,), jnp.float32),
            pltpu.VMEM((_CHP,), jnp.float32),
            pltpu.VMEM((_DSLAB,), jnp.float32),
            pltpu.VMEM((_DSLAB,), jnp.float32),
            pltpu.VMEM((_CHP,), jnp.float32),
        ],
    )
    return f(srcN, dstN, wN, zeros)


def _scale_rows(rows, ebuf):
    for g in range(_NGRP_FULL):
        nvv = lax.bitcast_convert_type(ebuf[2, pl.ds(g * 16, 16)],
                                       jnp.float32)
        for k in range(16):
            nv = nvv[k]
            r = g * 16 + k
            for j in range(_D // 16):
                sl = pl.ds(j * 16, 16)
                rows[r, sl] = rows[r, sl] * nv
    if _TAIL:
        nvt = lax.bitcast_convert_type(ebuf[2, pl.ds(_CH - 16, 16)],
                                       jnp.float32)
        for k in range(_TAIL):
            nv = nvt[16 - _TAIL + k]
            r = 16 * _NGRP_FULL + k
            for j in range(_D // 16):
                sl = pl.ds(j * 16, 16)
                rows[r, sl] = rows[r, sl] * nv


def _spmm_body(x, edata, zeros, p_out, acc, rows_a, rows_b, ebuf_a, ebuf_b,
               gsem_a, gsem_b, isem_a, isem_b, ssem_a, ssem_b):
    c = lax.axis_index("c")
    s = lax.axis_index("s")
    t = c * 16 + s
    c0 = t * _CPT
    slot_a = (rows_a, ebuf_a, gsem_a, isem_a, ssem_a)
    slot_b = (rows_b, ebuf_b, gsem_b, isem_b, ssem_b)

    pltpu.sync_copy(edata.at[pl.ds(8 * c0, 8)], ebuf_a)
    pltpu.async_copy(x.at[ebuf_a.at[0]], rows_a, gsem_a)
    pltpu.sync_copy(zeros, acc.at[pl.ds(s * _SLAB, _SLAB)])
    plsc.subcore_barrier()

    def do(i, slot, slot_o, wait_prev, has_next):
        rows, ebuf, gsem, isem, ssem = slot
        rows_o, ebuf_o, gsem_o, isem_o, ssem_o = slot_o

        def _maybe(pred, fn):
            if pred is True:
                fn()
            else:
                pl.when(pred)(fn)

        _maybe(wait_prev, lambda: pltpu.make_async_copy(
            rows_o, acc.at[ebuf_o.at[1]], ssem_o).wait())

        def prefetch_idx():
            pltpu.async_copy(edata.at[pl.ds(8 * (i + 1), 8)], ebuf_o, isem_o)

        _maybe(has_next, prefetch_idx)
        pltpu.make_async_copy(x.at[ebuf.at[0]], rows, gsem).wait()

        def next_gather():
            pltpu.make_async_copy(edata.at[pl.ds(8 * (i + 1), 8)], ebuf_o,
                                  isem_o).wait()
            pltpu.async_copy(x.at[ebuf_o.at[0]], rows_o, gsem_o)

        _maybe(has_next, next_gather)
        _scale_rows(rows, ebuf)
        pltpu.async_copy(rows, acc.at[ebuf.at[1]], ssem, add=True)

    def pair(p, carry):
        i = c0 + 2 * p
        do(i, slot_a, slot_b, wait_prev=(p > 0), has_next=True)
        do(i + 1, slot_b, slot_a, wait_prev=True,
           has_next=(p < _CPT // 2 - 1))
        return carry

    lax.fori_loop(0, _CPT // 2, pair, 0)
    pltpu.make_async_copy(rows_b, acc.at[ebuf_b.at[1]], ssem_b).wait()
    plsc.subcore_barrier()
    pltpu.sync_copy(acc.at[pl.ds(s * _SLAB, _SLAB)], p_out.at[c, s])


def _run_spmm(x, edata, zeros):
    f = pl.kernel(
        _spmm_body,
        out_type=jax.ShapeDtypeStruct((2, 16, _SLAB, _D), jnp.float32),
        mesh=_mesh(),
        compiler_params=pltpu.CompilerParams(needs_layout_passes=False),
        scratch_types=[
            pltpu.VMEM_SHARED((_N, _D), jnp.float32),
            pltpu.VMEM((_CH, _D), jnp.float32),
            pltpu.VMEM((_CH, _D), jnp.float32),
            pltpu.VMEM((8, _CH), jnp.int32),
            pltpu.VMEM((8, _CH), jnp.int32),
            pltpu.SemaphoreType.DMA,
            pltpu.SemaphoreType.DMA,
            pltpu.SemaphoreType.DMA,
            pltpu.SemaphoreType.DMA,
            pltpu.SemaphoreType.DMA,
            pltpu.SemaphoreType.DMA,
        ],
    )
    return f(x, edata, zeros)


_BLK = 1000


def _tc1_kern(x_ref, p0_ref, p1_ref, w0_ref, w1_ref, tx1_ref, acc_ref):
    tx1 = p0_ref[...] + p1_ref[...]
    tx1_ref[...] = tx1
    acc_ref[...] = (
        jnp.dot(x_ref[...], w0_ref[...], preferred_element_type=jnp.float32)
        + jnp.dot(tx1, w1_ref[...], preferred_element_type=jnp.float32))


def _tc1(x, p0, p1, w0, w1):
    grid = (_N // _BLK,)
    row = pl.BlockSpec((_BLK, _D), lambda i: (i, 0))
    full = pl.BlockSpec((_D, _D), lambda i: (0, 0))
    return pl.pallas_call(
        _tc1_kern,
        grid=grid,
        in_specs=[row, row, row, full, full],
        out_specs=[row, row],
        out_shape=[jax.ShapeDtypeStruct((_N, _D), jnp.float32),
                   jax.ShapeDtypeStruct((_N, _D), jnp.float32)],
    )(x, p0, p1, w0, w1)


def _tc2_kern(acc_ref, x_ref, q0_ref, q1_ref, w2_ref, out_ref):
    tx2 = 2.0 * (q0_ref[...] + q1_ref[...]) - x_ref[...]
    out_ref[...] = jnp.maximum(
        acc_ref[...]
        + jnp.dot(tx2, w2_ref[...], preferred_element_type=jnp.float32), 0.0)


def _tc2(acc, x, q0, q1, w2):
    grid = (_N // _BLK,)
    row = pl.BlockSpec((_BLK, _D), lambda i: (i, 0))
    full = pl.BlockSpec((_D, _D), lambda i: (0, 0))
    return pl.pallas_call(
        _tc2_kern,
        grid=grid,
        in_specs=[row, row, row, row, full],
        out_specs=row,
        out_shape=jax.ShapeDtypeStruct((_N, _D), jnp.float32),
    )(acc, x, q0, q1, w2)


_BN_INV = float(1.0 / np.sqrt(1.0 + 1e-5))


def _head_kern(x_ref, wc1_ref, bc1_ref, g_ref, b_ref, wc2_ref, bc2_ref,
               out_ref):
    h = jnp.maximum(
        jnp.dot(x_ref[...], wc1_ref[...], preferred_element_type=jnp.float32)
        + bc1_ref[...], 0.0)
    h = h * (g_ref[...] * _BN_INV) + b_ref[...]
    out_ref[...] = (
        jnp.dot(h, wc2_ref[...], preferred_element_type=jnp.float32)
        + bc2_ref[...])


def _head(x, wc1, bc1, bn_g, bn_b, wc2p, bc2p):
    grid = (_N // _BLK,)
    row = pl.BlockSpec((_BLK, _D), lambda i: (i, 0))
    return pl.pallas_call(
        _head_kern,
        grid=grid,
        in_specs=[
            row,
            pl.BlockSpec((_D, 256), lambda i: (0, 0)),
            pl.BlockSpec((1, 256), lambda i: (0, 0)),
            pl.BlockSpec((1, 256), lambda i: (0, 0)),
            pl.BlockSpec((1, 256), lambda i: (0, 0)),
            pl.BlockSpec((256, _D), lambda i: (0, 0)),
            pl.BlockSpec((1, _D), lambda i: (0, 0)),
        ],
        out_specs=pl.BlockSpec((_BLK, _D), lambda i: (i, 0)),
        out_shape=jax.ShapeDtypeStruct((_N, _D), jnp.float32),
    )(x, wc1, bc1, bn_g, bn_b, wc2p, bc2p)


def _build_edata(src, dst, norm_flat):
    srcp = src.reshape(_NCHUNK, 1, _CH)
    dstp = dst.reshape(_NCHUNK, 1, _CH)
    nrmp = lax.bitcast_convert_type(norm_flat, jnp.int32).reshape(
        _NCHUNK, 1, _CH)
    pad = jnp.zeros((_NCHUNK, 5, _CH), jnp.int32)
    return jnp.concatenate([srcp, dstp, nrmp, pad], axis=1).reshape(
        8 * _NCHUNK, _CH)


def kernel(features, edge_index, edge_weight, W0_0, W0_1, W0_2, W1_0, W1_1,
           W1_2, W2_0, W2_1, W2_2, Wc1, bc1, bn_g, bn_b, Wc2, bc2):
    src = edge_index[0]
    dst = edge_index[1]
    srcN = src.reshape(_NCHP, _CHP)
    dstN = dst.reshape(_NCHP, _CHP)
    wN = edge_weight.reshape(_NCHP, _CHP)
    zeros = jnp.zeros((_DSLAB,), jnp.float32)
    zeros_r = jnp.zeros((_SLAB, _D), jnp.float32)

    norm = _run_prep(srcN, dstN, wN, zeros)
    edata = _build_edata(src, dst, norm.reshape(-1))

    x = features
    for (w0, w1, w2_) in ((W0_0, W0_1, W0_2), (W1_0, W1_1, W1_2),
                          (W2_0, W2_1, W2_2)):
        p = _run_spmm(x, edata, zeros_r)
        tx1, acc = _tc1(x, p[0].reshape(_N, _D), p[1].reshape(_N, _D), w0, w1)
        q = _run_spmm(tx1, edata, zeros_r)
        x = _tc2(acc, x, q[0].reshape(_N, _D), q[1].reshape(_N, _D), w2_)

    wc2p = jnp.zeros((256, _D), jnp.float32).at[:, :Wc2.shape[1]].set(Wc2)
    bc2p = jnp.zeros((1, _D), jnp.float32).at[0, :bc2.shape[0]].set(bc2)
    logit_pad = _head(x, Wc1, bc1.reshape(1, 256), bn_g.reshape(1, 256),
                      bn_b.reshape(1, 256), wc2p, bc2p)
    return (logit_pad[:, :Wc2.shape[1]], edge_weight)


# speedup vs baseline: 1.2388x; 1.2388x over previous
import jax
import jax.numpy as jnp
import numpy as np
from jax import lax
from jax.experimental import pallas as pl
from jax.experimental.pallas import tpu as pltpu
from jax.experimental.pallas import tpu_sc as plsc

_N = 10000
_E = 320000
_D = 128
_CH = 100
_NCHUNK = _E // _CH
_CPT = _NCHUNK // 32
_SLAB = _N // 16
_NP = 10240
_DSLAB = _NP // 16
_NGRP_FULL = _CH // 16
_TAIL = _CH - 16 * _NGRP_FULL

_CHP = 500
_NCHP = _E // _CHP
_GSP = list(range(0, _CHP - 15, 16)) + ([_CHP - 16] if _CHP % 16 else [])


def _mesh():
    return plsc.VectorSubcoreMesh(core_axis_name="c", subcore_axis_name="s")


def _rsqrt16(d):
    pos = d > 0.0
    dc = jnp.where(pos, d, 1.0)
    i = lax.bitcast_convert_type(dc, jnp.int32)
    i = jnp.int32(0x5F3759DF) - lax.shift_right_arithmetic(i, 1)
    y = lax.bitcast_convert_type(i, jnp.float32)
    for _ in range(3):
        y = y * (1.5 - 0.5 * dc * y * y)
    return jnp.where(pos, y, 0.0)


def _prep_body(srcN, dstN, wN, zeros, norm_out, accD, dinvS, dinvb,
               srcb, dstb, wb, wzb, dbuf, obuf, nb):
    c = lax.axis_index("c")
    s = lax.axis_index("s")
    t = c * 16 + s

    pltpu.sync_copy(zeros, accD.at[pl.ds(s * _DSLAB, _DSLAB)])
    plsc.subcore_barrier()

    def dchunk(i, carry):
        cid = s * (_NCHP // 16) + i
        pltpu.sync_copy(srcN.at[cid], srcb)
        pltpu.sync_copy(dstN.at[cid], dstb)
        pltpu.sync_copy(wN.at[cid], wb)
        for gs in _GSP:
            sl = pl.ds(gs, 16)
            wzb[sl] = jnp.where(srcb[sl] == dstb[sl], 0.0, wb[sl])
        pltpu.sync_copy(wzb, accD.at[srcb], add=True)
        return carry

    lax.fori_loop(0, _NCHP // 16, dchunk, 0)
    plsc.subcore_barrier()

    pltpu.sync_copy(accD.at[pl.ds(s * _DSLAB, _DSLAB)], dbuf)
    for k in range(_DSLAB // 16):
        sl = pl.ds(k * 16, 16)
        obuf[sl] = _rsqrt16(dbuf[sl])
    pltpu.sync_copy(obuf, dinvS.at[pl.ds(s * _DSLAB, _DSLAB)])
    plsc.subcore_barrier()

    pltpu.sync_copy(dinvS, dinvb)

    def nchunk(i, carry):
        cid = t * (_NCHP // 32) + i
        pltpu.sync_copy(srcN.at[cid], srcb)
        pltpu.sync_copy(dstN.at[cid], dstb)
        pltpu.sync_copy(wN.at[cid], wb)
        for gs in _GSP:
            sl = pl.ds(gs, 16)
            sv = srcb[sl]
            dv = dstb[sl]
            wz = jnp.where(sv == dv, 0.0, wb[sl])
            ds_ = plsc.load_gather(dinvb, [sv])
            dd_ = plsc.load_gather(dinvb, [dv])
            nb[sl] = -(ds_ * wz * dd_)
        pltpu.sync_copy(nb, norm_out.at[cid])
        return carry

    lax.fori_loop(0, _NCHP // 32, nchunk, 0)


def _run_prep(srcN, dstN, wN, zeros):
    f = pl.kernel(
        _prep_body,
        out_type=jax.ShapeDtypeStruct((_NCHP, _CHP), jnp.float32),
        mesh=_mesh(),
        compiler_params=pltpu.CompilerParams(needs_layout_passes=False),
        scratch_types=[
            pltpu.VMEM_SHARED((_NP,), jnp.float32),
            pltpu.VMEM_SHARED((_NP,), jnp.float32),
            pltpu.VMEM((_NP,), jnp.float32),
            pltpu.VMEM((_CHP,), jnp.int32),
            pltpu.VMEM((_CHP,), jnp.int32),
            pltpu.VMEM((_CHP,), jnp.float32),
            pltpu.VMEM((_CHP,), jnp.float32),
            pltpu.VMEM((_DSLAB,), jnp.float32),
            pltpu.VMEM((_DSLAB,), jnp.float32),
            pltpu.VMEM((_CHP,), jnp.float32),
        ],
    )
    return f(srcN, dstN, wN, zeros)


def _scale_rows(rows, ebuf):
    def grp(g, carry2):
        nvv = lax.bitcast_convert_type(ebuf[2, pl.ds(g * 16, 16)],
                                       jnp.float32)
        for k in range(16):
            nv = nvv[k]
            r = g * 16 + k
            for j in range(_D // 16):
                sl = pl.ds(j * 16, 16)
                rows[r, sl] = rows[r, sl] * nv
        return carry2

    lax.fori_loop(0, _NGRP_FULL, grp, 0)
    if _TAIL:
        nvt = lax.bitcast_convert_type(ebuf[2, pl.ds(_CH - 16, 16)],
                                       jnp.float32)
        for k in range(_TAIL):
            nv = nvt[16 - _TAIL + k]
            r = 16 * _NGRP_FULL + k
            for j in range(_D // 16):
                sl = pl.ds(j * 16, 16)
                rows[r, sl] = rows[r, sl] * nv


def _spmm_body(x, edata, zeros, p_out, acc, rows_a, rows_b, ebuf_a, ebuf_b,
               gsem_a, gsem_b, isem_a, isem_b, ssem_a, ssem_b):
    c = lax.axis_index("c")
    s = lax.axis_index("s")
    t = c * 16 + s
    c0 = t * _CPT
    slot_a = (rows_a, ebuf_a, gsem_a, isem_a, ssem_a)
    slot_b = (rows_b, ebuf_b, gsem_b, isem_b, ssem_b)

    pltpu.sync_copy(edata.at[pl.ds(8 * c0, 8)], ebuf_a)
    pltpu.async_copy(x.at[ebuf_a.at[0]], rows_a, gsem_a)
    pltpu.sync_copy(zeros, acc.at[pl.ds(s * _SLAB, _SLAB)])
    plsc.subcore_barrier()

    def do(i, slot, slot_o, wait_prev, has_next):
        rows, ebuf, gsem, isem, ssem = slot
        rows_o, ebuf_o, gsem_o, isem_o, ssem_o = slot_o

        def _maybe(pred, fn):
            if pred is True:
                fn()
            else:
                pl.when(pred)(fn)

        _maybe(wait_prev, lambda: pltpu.make_async_copy(
            rows_o, acc.at[ebuf_o.at[1]], ssem_o).wait())

        def prefetch_idx():
            pltpu.async_copy(edata.at[pl.ds(8 * (i + 1), 8)], ebuf_o, isem_o)

        _maybe(has_next, prefetch_idx)
        pltpu.make_async_copy(x.at[ebuf.at[0]], rows, gsem).wait()

        def next_gather():
            pltpu.make_async_copy(edata.at[pl.ds(8 * (i + 1), 8)], ebuf_o,
                                  isem_o).wait()
            pltpu.async_copy(x.at[ebuf_o.at[0]], rows_o, gsem_o)

        _maybe(has_next, next_gather)
        _scale_rows(rows, ebuf)
        pltpu.async_copy(rows, acc.at[ebuf.at[1]], ssem, add=True)

    def pair(p, carry):
        i = c0 + 2 * p
        do(i, slot_a, slot_b, wait_prev=(p > 0), has_next=True)
        do(i + 1, slot_b, slot_a, wait_prev=True,
           has_next=(p < _CPT // 2 - 1))
        return carry

    lax.fori_loop(0, _CPT // 2, pair, 0)
    pltpu.make_async_copy(rows_b, acc.at[ebuf_b.at[1]], ssem_b).wait()
    plsc.subcore_barrier()
    pltpu.sync_copy(acc.at[pl.ds(s * _SLAB, _SLAB)], p_out.at[c, s])


def _run_spmm(x, edata, zeros):
    f = pl.kernel(
        _spmm_body,
        out_type=jax.ShapeDtypeStruct((2, 16, _SLAB, _D), jnp.float32),
        mesh=_mesh(),
        compiler_params=pltpu.CompilerParams(needs_layout_passes=False),
        scratch_types=[
            pltpu.VMEM_SHARED((_N, _D), jnp.float32),
            pltpu.VMEM((_CH, _D), jnp.float32),
            pltpu.VMEM((_CH, _D), jnp.float32),
            pltpu.VMEM((8, _CH), jnp.int32),
            pltpu.VMEM((8, _CH), jnp.int32),
            pltpu.SemaphoreType.DMA,
            pltpu.SemaphoreType.DMA,
            pltpu.SemaphoreType.DMA,
            pltpu.SemaphoreType.DMA,
            pltpu.SemaphoreType.DMA,
            pltpu.SemaphoreType.DMA,
        ],
    )
    return f(x, edata, zeros)


_BLK = 1000


def _tc1_kern(x_ref, p0_ref, p1_ref, w0_ref, w1_ref, tx1_ref, acc_ref):
    tx1 = p0_ref[...] + p1_ref[...]
    tx1_ref[...] = tx1
    acc_ref[...] = (
        jnp.dot(x_ref[...], w0_ref[...], preferred_element_type=jnp.float32)
        + jnp.dot(tx1, w1_ref[...], preferred_element_type=jnp.float32))


def _tc1(x, p0, p1, w0, w1):
    grid = (_N // _BLK,)
    row = pl.BlockSpec((_BLK, _D), lambda i: (i, 0))
    full = pl.BlockSpec((_D, _D), lambda i: (0, 0))
    return pl.pallas_call(
        _tc1_kern,
        grid=grid,
        in_specs=[row, row, row, full, full],
        out_specs=[row, row],
        out_shape=[jax.ShapeDtypeStruct((_N, _D), jnp.float32),
                   jax.ShapeDtypeStruct((_N, _D), jnp.float32)],
    )(x, p0, p1, w0, w1)


def _tc2_kern(acc_ref, x_ref, q0_ref, q1_ref, w2_ref, out_ref):
    tx2 = 2.0 * (q0_ref[...] + q1_ref[...]) - x_ref[...]
    out_ref[...] = jnp.maximum(
        acc_ref[...]
        + jnp.dot(tx2, w2_ref[...], preferred_element_type=jnp.float32), 0.0)


def _tc2(acc, x, q0, q1, w2):
    grid = (_N // _BLK,)
    row = pl.BlockSpec((_BLK, _D), lambda i: (i, 0))
    full = pl.BlockSpec((_D, _D), lambda i: (0, 0))
    return pl.pallas_call(
        _tc2_kern,
        grid=grid,
        in_specs=[row, row, row, row, full],
        out_specs=row,
        out_shape=jax.ShapeDtypeStruct((_N, _D), jnp.float32),
    )(acc, x, q0, q1, w2)


_BN_INV = float(1.0 / np.sqrt(1.0 + 1e-5))


def _head_kern(x_ref, wc1_ref, bc1_ref, g_ref, b_ref, wc2_ref, bc2_ref,
               out_ref):
    h = jnp.maximum(
        jnp.dot(x_ref[...], wc1_ref[...], preferred_element_type=jnp.float32)
        + bc1_ref[...], 0.0)
    h = h * (g_ref[...] * _BN_INV) + b_ref[...]
    out_ref[...] = (
        jnp.dot(h, wc2_ref[...], preferred_element_type=jnp.float32)
        + bc2_ref[...])


def _head(x, wc1, bc1, bn_g, bn_b, wc2p, bc2p):
    grid = (_N // _BLK,)
    row = pl.BlockSpec((_BLK, _D), lambda i: (i, 0))
    return pl.pallas_call(
        _head_kern,
        grid=grid,
        in_specs=[
            row,
            pl.BlockSpec((_D, 256), lambda i: (0, 0)),
            pl.BlockSpec((1, 256), lambda i: (0, 0)),
            pl.BlockSpec((1, 256), lambda i: (0, 0)),
            pl.BlockSpec((1, 256), lambda i: (0, 0)),
            pl.BlockSpec((256, _D), lambda i: (0, 0)),
            pl.BlockSpec((1, _D), lambda i: (0, 0)),
        ],
        out_specs=pl.BlockSpec((_BLK, _D), lambda i: (i, 0)),
        out_shape=jax.ShapeDtypeStruct((_N, _D), jnp.float32),
    )(x, wc1, bc1, bn_g, bn_b, wc2p, bc2p)


def _build_edata(src, dst, norm_flat):
    srcp = src.reshape(_NCHUNK, 1, _CH)
    dstp = dst.reshape(_NCHUNK, 1, _CH)
    nrmp = lax.bitcast_convert_type(norm_flat, jnp.int32).reshape(
        _NCHUNK, 1, _CH)
    pad = jnp.zeros((_NCHUNK, 5, _CH), jnp.int32)
    return jnp.concatenate([srcp, dstp, nrmp, pad], axis=1).reshape(
        8 * _NCHUNK, _CH)


def kernel(features, edge_index, edge_weight, W0_0, W0_1, W0_2, W1_0, W1_1,
           W1_2, W2_0, W2_1, W2_2, Wc1, bc1, bn_g, bn_b, Wc2, bc2):
    src = edge_index[0]
    dst = edge_index[1]
    srcN = src.reshape(_NCHP, _CHP)
    dstN = dst.reshape(_NCHP, _CHP)
    wN = edge_weight.reshape(_NCHP, _CHP)
    zeros = jnp.zeros((_DSLAB,), jnp.float32)
    zeros_r = jnp.zeros((_SLAB, _D), jnp.float32)

    norm = _run_prep(srcN, dstN, wN, zeros)
    edata = _build_edata(src, dst, norm.reshape(-1))

    x = features
    for (w0, w1, w2_) in ((W0_0, W0_1, W0_2), (W1_0, W1_1, W1_2),
                          (W2_0, W2_1, W2_2)):
        p = _run_spmm(x, edata, zeros_r)
        tx1, acc = _tc1(x, p[0].reshape(_N, _D), p[1].reshape(_N, _D), w0, w1)
        q = _run_spmm(tx1, edata, zeros_r)
        x = _tc2(acc, x, q[0].reshape(_N, _D), q[1].reshape(_N, _D), w2_)

    wc2p = jnp.zeros((256, _D), jnp.float32).at[:, :Wc2.shape[1]].set(Wc2)
    bc2p = jnp.zeros((1, _D), jnp.float32).at[0, :bc2.shape[0]].set(bc2)
    logit_pad = _head(x, Wc1, bc1.reshape(1, 256), bn_g.reshape(1, 256),
                      bn_b.reshape(1, 256), wc2p, bc2p)
    return (logit_pad[:, :Wc2.shape[1]], edge_weight)


# speedup vs baseline: 1.3056x; 1.0539x over previous
import jax
import jax.numpy as jnp
import numpy as np
from jax import lax
from jax.experimental import pallas as pl
from jax.experimental.pallas import tpu as pltpu
from jax.experimental.pallas import tpu_sc as plsc

_N = 10000
_E = 320000
_D = 128
_CH = 125
_NCHUNK = _E // _CH
_CPT = _NCHUNK // 32
_SLAB = _N // 16
_NP = 10240
_DSLAB = _NP // 16
_NGRP_FULL = _CH // 16
_TAIL = _CH - 16 * _NGRP_FULL

_CHP = 500
_NCHP = _E // _CHP
_GSP = list(range(0, _CHP - 15, 16)) + ([_CHP - 16] if _CHP % 16 else [])


def _mesh():
    return plsc.VectorSubcoreMesh(core_axis_name="c", subcore_axis_name="s")


def _rsqrt16(d):
    pos = d > 0.0
    dc = jnp.where(pos, d, 1.0)
    i = lax.bitcast_convert_type(dc, jnp.int32)
    i = jnp.int32(0x5F3759DF) - lax.shift_right_arithmetic(i, 1)
    y = lax.bitcast_convert_type(i, jnp.float32)
    for _ in range(3):
        y = y * (1.5 - 0.5 * dc * y * y)
    return jnp.where(pos, y, 0.0)


def _prep_body(srcN, dstN, wN, zeros, norm_out, accD, dinvS, dinvb,
               srcb, dstb, wb, wzb, dbuf, obuf, nb):
    c = lax.axis_index("c")
    s = lax.axis_index("s")
    t = c * 16 + s

    pltpu.sync_copy(zeros, accD.at[pl.ds(s * _DSLAB, _DSLAB)])
    plsc.subcore_barrier()

    def dchunk(i, carry):
        cid = s * (_NCHP // 16) + i
        pltpu.sync_copy(srcN.at[cid], srcb)
        pltpu.sync_copy(dstN.at[cid], dstb)
        pltpu.sync_copy(wN.at[cid], wb)
        for gs in _GSP:
            sl = pl.ds(gs, 16)
            wzb[sl] = jnp.where(srcb[sl] == dstb[sl], 0.0, wb[sl])
        pltpu.sync_copy(wzb, accD.at[srcb], add=True)
        return carry

    lax.fori_loop(0, _NCHP // 16, dchunk, 0)
    plsc.subcore_barrier()

    pltpu.sync_copy(accD.at[pl.ds(s * _DSLAB, _DSLAB)], dbuf)
    for k in range(_DSLAB // 16):
        sl = pl.ds(k * 16, 16)
        obuf[sl] = _rsqrt16(dbuf[sl])
    pltpu.sync_copy(obuf, dinvS.at[pl.ds(s * _DSLAB, _DSLAB)])
    plsc.subcore_barrier()

    pltpu.sync_copy(dinvS, dinvb)

    def nchunk(i, carry):
        cid = t * (_NCHP // 32) + i
        pltpu.sync_copy(srcN.at[cid], srcb)
        pltpu.sync_copy(dstN.at[cid], dstb)
        pltpu.sync_copy(wN.at[cid], wb)
        for gs in _GSP:
            sl = pl.ds(gs, 16)
            sv = srcb[sl]
            dv = dstb[sl]
            wz = jnp.where(sv == dv, 0.0, wb[sl])
            ds_ = plsc.load_gather(dinvb, [sv])
            dd_ = plsc.load_gather(dinvb, [dv])
            nb[sl] = -(ds_ * wz * dd_)
        pltpu.sync_copy(nb, norm_out.at[cid])
        return carry

    lax.fori_loop(0, _NCHP // 32, nchunk, 0)


def _run_prep(srcN, dstN, wN, zeros):
    f = pl.kernel(
        _prep_body,
        out_type=jax.ShapeDtypeStruct((_NCHP, _CHP), jnp.float32),
        mesh=_mesh(),
        compiler_params=pltpu.CompilerParams(needs_layout_passes=False),
        scratch_types=[
            pltpu.VMEM_SHARED((_NP,), jnp.float32),
            pltpu.VMEM_SHARED((_NP,), jnp.float32),
            pltpu.VMEM((_NP,), jnp.float32),
            pltpu.VMEM((_CHP,), jnp.int32),
            pltpu.VMEM((_CHP,), jnp.int32),
            pltpu.VMEM((_CHP,), jnp.float32),
            pltpu.VMEM((_CHP,), jnp.float32),
            pltpu.VMEM((_DSLAB,), jnp.float32),
            pltpu.VMEM((_DSLAB,), jnp.float32),
            pltpu.VMEM((_CHP,), jnp.float32),
        ],
    )
    return f(srcN, dstN, wN, zeros)


def _scale_rows(rows, ebuf):
    def grp(g, carry2):
        nvv = lax.bitcast_convert_type(ebuf[2, pl.ds(g * 16, 16)],
                                       jnp.float32)
        for k in range(16):
            nv = nvv[k]
            r = g * 16 + k
            for j in range(_D // 16):
                sl = pl.ds(j * 16, 16)
                rows[r, sl] = rows[r, sl] * nv
        return carry2

    lax.fori_loop(0, _NGRP_FULL, grp, 0, unroll=2)
    if _TAIL:
        nvt = lax.bitcast_convert_type(ebuf[2, pl.ds(_CH - 16, 16)],
                                       jnp.float32)
        for k in range(_TAIL):
            nv = nvt[16 - _TAIL + k]
            r = 16 * _NGRP_FULL + k
            for j in range(_D // 16):
                sl = pl.ds(j * 16, 16)
                rows[r, sl] = rows[r, sl] * nv


def _spmm_body(x, edata, zeros, p_out, acc, rows_a, rows_b, ebuf_a, ebuf_b,
               gsem_a, gsem_b, isem_a, isem_b, ssem_a, ssem_b):
    c = lax.axis_index("c")
    s = lax.axis_index("s")
    t = c * 16 + s
    c0 = t * _CPT
    slot_a = (rows_a, ebuf_a, gsem_a, isem_a, ssem_a)
    slot_b = (rows_b, ebuf_b, gsem_b, isem_b, ssem_b)

    pltpu.sync_copy(edata.at[pl.ds(8 * c0, 8)], ebuf_a)
    pltpu.async_copy(x.at[ebuf_a.at[0]], rows_a, gsem_a)
    pltpu.sync_copy(zeros, acc.at[pl.ds(s * _SLAB, _SLAB)])
    plsc.subcore_barrier()

    def do(i, slot, slot_o, wait_prev, has_next):
        rows, ebuf, gsem, isem, ssem = slot
        rows_o, ebuf_o, gsem_o, isem_o, ssem_o = slot_o

        def _maybe(pred, fn):
            if pred is True:
                fn()
            else:
                pl.when(pred)(fn)

        _maybe(wait_prev, lambda: pltpu.make_async_copy(
            rows_o, acc.at[ebuf_o.at[1]], ssem_o).wait())

        def prefetch_idx():
            pltpu.async_copy(edata.at[pl.ds(8 * (i + 1), 8)], ebuf_o, isem_o)

        _maybe(has_next, prefetch_idx)
        pltpu.make_async_copy(x.at[ebuf.at[0]], rows, gsem).wait()

        def next_gather():
            pltpu.make_async_copy(edata.at[pl.ds(8 * (i + 1), 8)], ebuf_o,
                                  isem_o).wait()
            pltpu.async_copy(x.at[ebuf_o.at[0]], rows_o, gsem_o)

        _maybe(has_next, next_gather)
        _scale_rows(rows, ebuf)
        pltpu.async_copy(rows, acc.at[ebuf.at[1]], ssem, add=True)

    def pair(p, carry):
        i = c0 + 2 * p
        do(i, slot_a, slot_b, wait_prev=(p > 0), has_next=True)
        do(i + 1, slot_b, slot_a, wait_prev=True,
           has_next=(p < _CPT // 2 - 1))
        return carry

    lax.fori_loop(0, _CPT // 2, pair, 0)
    pltpu.make_async_copy(rows_b, acc.at[ebuf_b.at[1]], ssem_b).wait()
    plsc.subcore_barrier()
    pltpu.sync_copy(acc.at[pl.ds(s * _SLAB, _SLAB)], p_out.at[c, s])


def _run_spmm(x, edata, zeros):
    f = pl.kernel(
        _spmm_body,
        out_type=jax.ShapeDtypeStruct((2, 16, _SLAB, _D), jnp.float32),
        mesh=_mesh(),
        compiler_params=pltpu.CompilerParams(needs_layout_passes=False),
        scratch_types=[
            pltpu.VMEM_SHARED((_N, _D), jnp.float32),
            pltpu.VMEM((_CH, _D), jnp.float32),
            pltpu.VMEM((_CH, _D), jnp.float32),
            pltpu.VMEM((8, _CH), jnp.int32),
            pltpu.VMEM((8, _CH), jnp.int32),
            pltpu.SemaphoreType.DMA,
            pltpu.SemaphoreType.DMA,
            pltpu.SemaphoreType.DMA,
            pltpu.SemaphoreType.DMA,
            pltpu.SemaphoreType.DMA,
            pltpu.SemaphoreType.DMA,
        ],
    )
    return f(x, edata, zeros)


_BLK = 1000


def _tc1_kern(x_ref, p0_ref, p1_ref, w0_ref, w1_ref, tx1_ref, acc_ref):
    tx1 = p0_ref[...] + p1_ref[...]
    tx1_ref[...] = tx1
    acc_ref[...] = (
        jnp.dot(x_ref[...], w0_ref[...], preferred_element_type=jnp.float32)
        + jnp.dot(tx1, w1_ref[...], preferred_element_type=jnp.float32))


def _tc1(x, p0, p1, w0, w1):
    grid = (_N // _BLK,)
    row = pl.BlockSpec((_BLK, _D), lambda i: (i, 0))
    full = pl.BlockSpec((_D, _D), lambda i: (0, 0))
    return pl.pallas_call(
        _tc1_kern,
        grid=grid,
        in_specs=[row, row, row, full, full],
        out_specs=[row, row],
        out_shape=[jax.ShapeDtypeStruct((_N, _D), jnp.float32),
                   jax.ShapeDtypeStruct((_N, _D), jnp.float32)],
    )(x, p0, p1, w0, w1)


def _tc2_kern(acc_ref, x_ref, q0_ref, q1_ref, w2_ref, out_ref):
    tx2 = 2.0 * (q0_ref[...] + q1_ref[...]) - x_ref[...]
    out_ref[...] = jnp.maximum(
        acc_ref[...]
        + jnp.dot(tx2, w2_ref[...], preferred_element_type=jnp.float32), 0.0)


def _tc2(acc, x, q0, q1, w2):
    grid = (_N // _BLK,)
    row = pl.BlockSpec((_BLK, _D), lambda i: (i, 0))
    full = pl.BlockSpec((_D, _D), lambda i: (0, 0))
    return pl.pallas_call(
        _tc2_kern,
        grid=grid,
        in_specs=[row, row, row, row, full],
        out_specs=row,
        out_shape=jax.ShapeDtypeStruct((_N, _D), jnp.float32),
    )(acc, x, q0, q1, w2)


_BN_INV = float(1.0 / np.sqrt(1.0 + 1e-5))


def _head_kern(x_ref, wc1_ref, bc1_ref, g_ref, b_ref, wc2_ref, bc2_ref,
               out_ref):
    h = jnp.maximum(
        jnp.dot(x_ref[...], wc1_ref[...], preferred_element_type=jnp.float32)
        + bc1_ref[...], 0.0)
    h = h * (g_ref[...] * _BN_INV) + b_ref[...]
    out_ref[...] = (
        jnp.dot(h, wc2_ref[...], preferred_element_type=jnp.float32)
        + bc2_ref[...])


def _head(x, wc1, bc1, bn_g, bn_b, wc2p, bc2p):
    grid = (_N // _BLK,)
    row = pl.BlockSpec((_BLK, _D), lambda i: (i, 0))
    return pl.pallas_call(
        _head_kern,
        grid=grid,
        in_specs=[
            row,
            pl.BlockSpec((_D, 256), lambda i: (0, 0)),
            pl.BlockSpec((1, 256), lambda i: (0, 0)),
            pl.BlockSpec((1, 256), lambda i: (0, 0)),
            pl.BlockSpec((1, 256), lambda i: (0, 0)),
            pl.BlockSpec((256, _D), lambda i: (0, 0)),
            pl.BlockSpec((1, _D), lambda i: (0, 0)),
        ],
        out_specs=pl.BlockSpec((_BLK, _D), lambda i: (i, 0)),
        out_shape=jax.ShapeDtypeStruct((_N, _D), jnp.float32),
    )(x, wc1, bc1, bn_g, bn_b, wc2p, bc2p)


def _build_edata(src, dst, norm_flat):
    srcp = src.reshape(_NCHUNK, 1, _CH)
    dstp = dst.reshape(_NCHUNK, 1, _CH)
    nrmp = lax.bitcast_convert_type(norm_flat, jnp.int32).reshape(
        _NCHUNK, 1, _CH)
    pad = jnp.zeros((_NCHUNK, 5, _CH), jnp.int32)
    return jnp.concatenate([srcp, dstp, nrmp, pad], axis=1).reshape(
        8 * _NCHUNK, _CH)


def kernel(features, edge_index, edge_weight, W0_0, W0_1, W0_2, W1_0, W1_1,
           W1_2, W2_0, W2_1, W2_2, Wc1, bc1, bn_g, bn_b, Wc2, bc2):
    src = edge_index[0]
    dst = edge_index[1]
    srcN = src.reshape(_NCHP, _CHP)
    dstN = dst.reshape(_NCHP, _CHP)
    wN = edge_weight.reshape(_NCHP, _CHP)
    zeros = jnp.zeros((_DSLAB,), jnp.float32)
    zeros_r = jnp.zeros((_SLAB, _D), jnp.float32)

    norm = _run_prep(srcN, dstN, wN, zeros)
    edata = _build_edata(src, dst, norm.reshape(-1))

    x = features
    for (w0, w1, w2_) in ((W0_0, W0_1, W0_2), (W1_0, W1_1, W1_2),
                          (W2_0, W2_1, W2_2)):
        p = _run_spmm(x, edata, zeros_r)
        tx1, acc = _tc1(x, p[0].reshape(_N, _D), p[1].reshape(_N, _D), w0, w1)
        q = _run_spmm(tx1, edata, zeros_r)
        x = _tc2(acc, x, q[0].reshape(_N, _D), q[1].reshape(_N, _D), w2_)

    wc2p = jnp.zeros((256, _D), jnp.float32).at[:, :Wc2.shape[1]].set(Wc2)
    bc2p = jnp.zeros((1, _D), jnp.float32).at[0, :bc2.shape[0]].set(bc2)
    logit_pad = _head(x, Wc1, bc1.reshape(1, 256), bn_g.reshape(1, 256),
                      bn_b.reshape(1, 256), wc2p, bc2p)
    return (logit_pad[:, :Wc2.shape[1]], edge_weight)
